# Initial kernel scaffold; baseline (speedup 1.0000x reference)
import jax
import jax.numpy as jnp
from jax.experimental import pallas as pl


def kernel(point_xyz_few, point_feature_few, point_xyz_large, point_feature_large, W1, b1, g1, beta1, W2, b2, g2, beta2):
    raise NotImplementedError("write your pallas kernel here")


# speedup vs baseline: 16.4877x; 16.4877x over previous
import functools

import jax
import jax.numpy as jnp
from jax.experimental import pallas as pl


def _nn_interp_l1_kernel(xl_ref, xf_ref, ff_ref, fl_ref, w1a_ref, w1b_ref,
                         b1_ref, y1_ref, st_ref, *, S):
    xl = xl_ref[0]
    xf = xf_ref[0]
    d = -2.0 * jax.lax.dot_general(
        xl.astype(jnp.bfloat16), xf.astype(jnp.bfloat16),
        (((1,), (1,)), ((), ())),
        preferred_element_type=jnp.float32)
    d = d + jnp.sum(xl * xl, axis=1, keepdims=True)
    d = d + jnp.sum(xf * xf, axis=1)[None, :]
    iota = jax.lax.broadcasted_iota(jnp.int32, d.shape, 1)
    wacc = jnp.zeros(d.shape, jnp.float32)
    norm = jnp.zeros((d.shape[0], 1), jnp.float32)
    for _ in range(3):
        m = jnp.min(d, axis=1, keepdims=True)
        cand = jnp.where(d == m, iota, S)
        j = jnp.min(cand, axis=1, keepdims=True)
        onehot = iota == j
        w = 1.0 / (m + 1e-8)
        wacc = wacc + jnp.where(onehot, w, 0.0)
        norm = norm + w
        d = jnp.where(onehot, jnp.float32(jnp.inf), d)
    wmat = wacc / norm
    interp = jnp.dot(wmat, ff_ref[0],
                     preferred_element_type=jnp.float32)
    y1 = jax.lax.dot_general(w1a_ref[...].astype(jnp.bfloat16),
                             fl_ref[0].astype(jnp.bfloat16),
                             (((1,), (1,)), ((), ())),
                             preferred_element_type=jnp.float32)
    y1 = y1 + jax.lax.dot_general(w1b_ref[...].astype(jnp.bfloat16),
                                  interp.astype(jnp.bfloat16),
                                  (((1,), (1,)), ((), ())),
                                  preferred_element_type=jnp.float32)
    y1 = y1 + b1_ref[...]
    y1_ref[0] = y1
    st_ref[0, 0, :] = jnp.sum(y1, axis=1)
    st_ref[0, 1, :] = jnp.sum(y1 * y1, axis=1)


def _bn_mm_kernel(y_ref, a_ref, c_ref, w2_ref, b2_ref, y2_ref, st_ref):
    h = jnp.maximum(y_ref[0] * a_ref[...] + c_ref[...], 0.0)
    y2 = jnp.dot(w2_ref[...].astype(jnp.bfloat16), h.astype(jnp.bfloat16),
                 preferred_element_type=jnp.float32) + b2_ref[...]
    y2_ref[0] = y2
    st_ref[0, 0, :] = jnp.sum(y2, axis=1)
    st_ref[0, 1, :] = jnp.sum(y2 * y2, axis=1)


def _bn_out_kernel(y_ref, a_ref, c_ref, o_ref):
    o_ref[0] = jnp.maximum(y_ref[0] * a_ref[...] + c_ref[...], 0.0)


def _scale_shift(st, m, gamma, beta):
    s = jnp.sum(st, axis=0)
    mean = s[0] / m
    var = s[1] / m - mean * mean
    inv = jax.lax.rsqrt(var + 1e-5)
    a = gamma * inv
    c = beta - mean * a
    return a[:, None], c[:, None]


def kernel(point_xyz_few, point_feature_few, point_xyz_large,
           point_feature_large, W1, b1, g1, beta1, W2, b2, g2, beta2):
    Bn, S, _ = point_xyz_few.shape
    N = point_xyz_large.shape[1]
    C_FEW = point_feature_few.shape[2]
    C_LARGE = point_feature_large.shape[2]
    C_MID = W1.shape[0]
    C_OUT = W2.shape[0]
    TN = 256
    NT = N // TN
    M = Bn * N
    grid = (Bn, NT)

    y1, st1 = pl.pallas_call(
        functools.partial(_nn_interp_l1_kernel, S=S),
        grid=grid,
        in_specs=[
            pl.BlockSpec((1, TN, 3), lambda b, t: (b, t, 0)),
            pl.BlockSpec((1, S, 3), lambda b, t: (b, 0, 0)),
            pl.BlockSpec((1, S, C_FEW), lambda b, t: (b, 0, 0)),
            pl.BlockSpec((1, TN, C_LARGE), lambda b, t: (b, t, 0)),
            pl.BlockSpec((C_MID, C_LARGE), lambda b, t: (0, 0)),
            pl.BlockSpec((C_MID, C_FEW), lambda b, t: (0, 0)),
            pl.BlockSpec((C_MID, 1), lambda b, t: (0, 0)),
        ],
        out_specs=[
            pl.BlockSpec((1, C_MID, TN), lambda b, t: (b, 0, t)),
            pl.BlockSpec((1, 2, C_MID), lambda b, t: (b * NT + t, 0, 0)),
        ],
        out_shape=[
            jax.ShapeDtypeStruct((Bn, C_MID, N), jnp.float32),
            jax.ShapeDtypeStruct((Bn * NT, 2, C_MID), jnp.float32),
        ],
    )(point_xyz_large, point_xyz_few, point_feature_few, point_feature_large,
      W1[:, :C_LARGE], W1[:, C_LARGE:], b1[:, None])

    a1, c1 = _scale_shift(st1, M, g1, beta1)
    y2, st2 = pl.pallas_call(
        _bn_mm_kernel,
        grid=grid,
        in_specs=[
            pl.BlockSpec((1, C_MID, TN), lambda b, t: (b, 0, t)),
            pl.BlockSpec((C_MID, 1), lambda b, t: (0, 0)),
            pl.BlockSpec((C_MID, 1), lambda b, t: (0, 0)),
            pl.BlockSpec((C_OUT, C_MID), lambda b, t: (0, 0)),
            pl.BlockSpec((C_OUT, 1), lambda b, t: (0, 0)),
        ],
        out_specs=[
            pl.BlockSpec((1, C_OUT, TN), lambda b, t: (b, 0, t)),
            pl.BlockSpec((1, 2, C_OUT), lambda b, t: (b * NT + t, 0, 0)),
        ],
        out_shape=[
            jax.ShapeDtypeStruct((Bn, C_OUT, N), jnp.float32),
            jax.ShapeDtypeStruct((Bn * NT, 2, C_OUT), jnp.float32),
        ],
    )(y1, a1, c1, W2, b2[:, None])

    a2, c2 = _scale_shift(st2, M, g2, beta2)
    out = pl.pallas_call(
        _bn_out_kernel,
        grid=grid,
        in_specs=[
            pl.BlockSpec((1, C_OUT, TN), lambda b, t: (b, 0, t)),
            pl.BlockSpec((C_OUT, 1), lambda b, t: (0, 0)),
            pl.BlockSpec((C_OUT, 1), lambda b, t: (0, 0)),
        ],
        out_specs=pl.BlockSpec((1, C_OUT, TN), lambda b, t: (b, 0, t)),
        out_shape=jax.ShapeDtypeStruct((Bn, C_OUT, N), jnp.float32),
    )(y2, a2, c2)
    return out


# speedup vs baseline: 18.2474x; 1.1067x over previous
import functools

import jax
import jax.numpy as jnp
from jax.experimental import pallas as pl


def _nn_interp_l1_kernel(xl_ref, xf_ref, ff_ref, fl_ref, w1a_ref, w1b_ref,
                         b1_ref, y1_ref, st_ref, *, S):
    xl = xl_ref[0]
    xf = xf_ref[0]
    nl = jnp.sum(xl * xl, axis=1, keepdims=True)
    nf = jnp.sum(xf * xf, axis=1)[None, :]
    d = -2.0 * jax.lax.dot_general(
        xl.astype(jnp.bfloat16), xf.astype(jnp.bfloat16),
        (((1,), (1,)), ((), ())),
        preferred_element_type=jnp.float32)
    d = d + nl
    d = d + nf
    dorig = d
    inf = jnp.float32(jnp.inf)
    slots = jnp.full((d.shape[0], 1), 3.0, jnp.float32)
    norm = jnp.zeros((d.shape[0], 1), jnp.float32)
    for _ in range(3):
        m = jnp.min(d, axis=1, keepdims=True)
        eq = d == m
        cnt = jnp.sum(jnp.where(eq, 1.0, 0.0), axis=1, keepdims=True)
        take = jnp.minimum(cnt, slots)
        w = 1.0 / (m + 1e-8)
        norm = norm + w * take
        d = jnp.where(eq & (slots > 0.0), inf, d)
        slots = slots - take
    wmat = jnp.where(d == inf, (1.0 / (dorig + 1e-8)) / norm, 0.0)
    interp = jnp.dot(wmat, ff_ref[0],
                     preferred_element_type=jnp.float32)
    y1 = jax.lax.dot_general(w1a_ref[...].astype(jnp.bfloat16),
                             fl_ref[0].astype(jnp.bfloat16),
                             (((1,), (1,)), ((), ())),
                             preferred_element_type=jnp.float32)
    y1 = y1 + jax.lax.dot_general(w1b_ref[...].astype(jnp.bfloat16),
                                  interp.astype(jnp.bfloat16),
                                  (((1,), (1,)), ((), ())),
                                  preferred_element_type=jnp.float32)
    y1 = y1 + b1_ref[...]
    y1_ref[0] = y1
    st_ref[0, 0, :] = jnp.sum(y1, axis=1)
    st_ref[0, 1, :] = jnp.sum(y1 * y1, axis=1)


def _bn_mm_kernel(y_ref, a_ref, c_ref, w2_ref, b2_ref, y2_ref, st_ref):
    h = jnp.maximum(y_ref[0] * a_ref[...] + c_ref[...], 0.0)
    y2 = jnp.dot(w2_ref[...].astype(jnp.bfloat16), h.astype(jnp.bfloat16),
                 preferred_element_type=jnp.float32) + b2_ref[...]
    y2_ref[0] = y2
    st_ref[0, 0, :] = jnp.sum(y2, axis=1)
    st_ref[0, 1, :] = jnp.sum(y2 * y2, axis=1)


def _bn_out_kernel(y_ref, a_ref, c_ref, o_ref):
    o_ref[0] = jnp.maximum(y_ref[0] * a_ref[...] + c_ref[...], 0.0)


def _scale_shift(st, m, gamma, beta):
    s = jnp.sum(st, axis=0)
    mean = s[0] / m
    var = s[1] / m - mean * mean
    inv = jax.lax.rsqrt(var + 1e-5)
    a = gamma * inv
    c = beta - mean * a
    return a[:, None], c[:, None]


def kernel(point_xyz_few, point_feature_few, point_xyz_large,
           point_feature_large, W1, b1, g1, beta1, W2, b2, g2, beta2):
    Bn, S, _ = point_xyz_few.shape
    N = point_xyz_large.shape[1]
    C_FEW = point_feature_few.shape[2]
    C_LARGE = point_feature_large.shape[2]
    C_MID = W1.shape[0]
    C_OUT = W2.shape[0]
    TN = 256
    NT = N // TN
    M = Bn * N
    grid = (Bn, NT)

    y1, st1 = pl.pallas_call(
        functools.partial(_nn_interp_l1_kernel, S=S),
        grid=grid,
        in_specs=[
            pl.BlockSpec((1, TN, 3), lambda b, t: (b, t, 0)),
            pl.BlockSpec((1, S, 3), lambda b, t: (b, 0, 0)),
            pl.BlockSpec((1, S, C_FEW), lambda b, t: (b, 0, 0)),
            pl.BlockSpec((1, TN, C_LARGE), lambda b, t: (b, t, 0)),
            pl.BlockSpec((C_MID, C_LARGE), lambda b, t: (0, 0)),
            pl.BlockSpec((C_MID, C_FEW), lambda b, t: (0, 0)),
            pl.BlockSpec((C_MID, 1), lambda b, t: (0, 0)),
        ],
        out_specs=[
            pl.BlockSpec((1, C_MID, TN), lambda b, t: (b, 0, t)),
            pl.BlockSpec((1, 2, C_MID), lambda b, t: (b * NT + t, 0, 0)),
        ],
        out_shape=[
            jax.ShapeDtypeStruct((Bn, C_MID, N), jnp.float32),
            jax.ShapeDtypeStruct((Bn * NT, 2, C_MID), jnp.float32),
        ],
    )(point_xyz_large, point_xyz_few, point_feature_few, point_feature_large,
      W1[:, :C_LARGE], W1[:, C_LARGE:], b1[:, None])

    a1, c1 = _scale_shift(st1, M, g1, beta1)
    y2, st2 = pl.pallas_call(
        _bn_mm_kernel,
        grid=grid,
        in_specs=[
            pl.BlockSpec((1, C_MID, TN), lambda b, t: (b, 0, t)),
            pl.BlockSpec((C_MID, 1), lambda b, t: (0, 0)),
            pl.BlockSpec((C_MID, 1), lambda b, t: (0, 0)),
            pl.BlockSpec((C_OUT, C_MID), lambda b, t: (0, 0)),
            pl.BlockSpec((C_OUT, 1), lambda b, t: (0, 0)),
        ],
        out_specs=[
            pl.BlockSpec((1, C_OUT, TN), lambda b, t: (b, 0, t)),
            pl.BlockSpec((1, 2, C_OUT), lambda b, t: (b * NT + t, 0, 0)),
        ],
        out_shape=[
            jax.ShapeDtypeStruct((Bn, C_OUT, N), jnp.float32),
            jax.ShapeDtypeStruct((Bn * NT, 2, C_OUT), jnp.float32),
        ],
    )(y1, a1, c1, W2, b2[:, None])

    a2, c2 = _scale_shift(st2, M, g2, beta2)
    out = pl.pallas_call(
        _bn_out_kernel,
        grid=grid,
        in_specs=[
            pl.BlockSpec((1, C_OUT, TN), lambda b, t: (b, 0, t)),
            pl.BlockSpec((C_OUT, 1), lambda b, t: (0, 0)),
            pl.BlockSpec((C_OUT, 1), lambda b, t: (0, 0)),
        ],
        out_specs=pl.BlockSpec((1, C_OUT, TN), lambda b, t: (b, 0, t)),
        out_shape=jax.ShapeDtypeStruct((Bn, C_OUT, N), jnp.float32),
    )(y2, a2, c2)
    return out


# speedup vs baseline: 25.7192x; 1.4095x over previous
import functools

import jax
import jax.numpy as jnp
from jax.experimental import pallas as pl


def _nn_interp_l1_kernel(xl_ref, xf_ref, ff_ref, fl_ref, w1a_ref, w1b_ref,
                         b1_ref, y1_ref, st_ref, *, S):
    xl = xl_ref[0]
    xf = xf_ref[0]
    nl = jnp.sum(xl * xl, axis=1, keepdims=True)
    nf = jnp.sum(xf * xf, axis=1)[None, :]
    d = -2.0 * jax.lax.dot_general(
        xl.astype(jnp.bfloat16), xf.astype(jnp.bfloat16),
        (((1,), (1,)), ((), ())),
        preferred_element_type=jnp.float32)
    d = d + nl
    d = d + nf
    dorig = d
    inf = jnp.float32(jnp.inf)
    slots = jnp.full((d.shape[0], 1), 3.0, jnp.float32)
    norm = jnp.zeros((d.shape[0], 1), jnp.float32)
    for _ in range(3):
        m = jnp.min(d, axis=1, keepdims=True)
        eq = d == m
        cnt = jnp.sum(jnp.where(eq, 1.0, 0.0), axis=1, keepdims=True)
        take = jnp.minimum(cnt, slots)
        w = 1.0 / (m + 1e-8)
        norm = norm + w * take
        d = jnp.where(eq & (slots > 0.0), inf, d)
        slots = slots - take
    wmat = jnp.where(d == inf, (1.0 / (dorig + 1e-8)) / norm, 0.0)
    interp = jnp.dot(wmat, ff_ref[0],
                     preferred_element_type=jnp.float32)
    y1 = jax.lax.dot_general(w1a_ref[...].astype(jnp.bfloat16),
                             fl_ref[0].astype(jnp.bfloat16),
                             (((1,), (1,)), ((), ())),
                             preferred_element_type=jnp.float32)
    y1 = y1 + jax.lax.dot_general(w1b_ref[...].astype(jnp.bfloat16),
                                  interp.astype(jnp.bfloat16),
                                  (((1,), (1,)), ((), ())),
                                  preferred_element_type=jnp.float32)
    y1 = y1 + b1_ref[...]
    y1_ref[0] = y1
    st_ref[0, 0, :] = jnp.sum(y1, axis=1)
    st_ref[0, 1, :] = jnp.sum(y1 * y1, axis=1)


def _bn_mm_kernel(y_ref, a_ref, c_ref, w2_ref, b2_ref, y2_ref, st_ref):
    h = jnp.maximum(y_ref[0] * a_ref[...] + c_ref[...], 0.0)
    y2 = jnp.dot(w2_ref[...].astype(jnp.bfloat16), h.astype(jnp.bfloat16),
                 preferred_element_type=jnp.float32) + b2_ref[...]
    y2_ref[0] = y2
    st_ref[0, 0, :] = jnp.sum(y2, axis=1)
    st_ref[0, 1, :] = jnp.sum(y2 * y2, axis=1)


def _bn_out_kernel(y_ref, a_ref, c_ref, o_ref):
    o_ref[0] = jnp.maximum(y_ref[0] * a_ref[...] + c_ref[...], 0.0)


def _scale_shift(st, m, gamma, beta):
    s = jnp.sum(st, axis=0)
    mean = s[0] / m
    var = s[1] / m - mean * mean
    inv = jax.lax.rsqrt(var + 1e-5)
    a = gamma * inv
    c = beta - mean * a
    return a[:, None], c[:, None]


def kernel(point_xyz_few, point_feature_few, point_xyz_large,
           point_feature_large, W1, b1, g1, beta1, W2, b2, g2, beta2):
    Bn, S, _ = point_xyz_few.shape
    N = point_xyz_large.shape[1]
    C_FEW = point_feature_few.shape[2]
    C_LARGE = point_feature_large.shape[2]
    C_MID = W1.shape[0]
    C_OUT = W2.shape[0]
    TN = 512
    NT = N // TN
    M = Bn * N
    grid = (Bn, NT)

    y1, st1 = pl.pallas_call(
        functools.partial(_nn_interp_l1_kernel, S=S),
        grid=grid,
        in_specs=[
            pl.BlockSpec((1, TN, 3), lambda b, t: (b, t, 0)),
            pl.BlockSpec((1, S, 3), lambda b, t: (b, 0, 0)),
            pl.BlockSpec((1, S, C_FEW), lambda b, t: (b, 0, 0)),
            pl.BlockSpec((1, TN, C_LARGE), lambda b, t: (b, t, 0)),
            pl.BlockSpec((C_MID, C_LARGE), lambda b, t: (0, 0)),
            pl.BlockSpec((C_MID, C_FEW), lambda b, t: (0, 0)),
            pl.BlockSpec((C_MID, 1), lambda b, t: (0, 0)),
        ],
        out_specs=[
            pl.BlockSpec((1, C_MID, TN), lambda b, t: (b, 0, t)),
            pl.BlockSpec((1, 2, C_MID), lambda b, t: (b * NT + t, 0, 0)),
        ],
        out_shape=[
            jax.ShapeDtypeStruct((Bn, C_MID, N), jnp.float32),
            jax.ShapeDtypeStruct((Bn * NT, 2, C_MID), jnp.float32),
        ],
    )(point_xyz_large, point_xyz_few, point_feature_few, point_feature_large,
      W1[:, :C_LARGE], W1[:, C_LARGE:], b1[:, None])

    a1, c1 = _scale_shift(st1, M, g1, beta1)
    y2, st2 = pl.pallas_call(
        _bn_mm_kernel,
        grid=grid,
        in_specs=[
            pl.BlockSpec((1, C_MID, TN), lambda b, t: (b, 0, t)),
            pl.BlockSpec((C_MID, 1), lambda b, t: (0, 0)),
            pl.BlockSpec((C_MID, 1), lambda b, t: (0, 0)),
            pl.BlockSpec((C_OUT, C_MID), lambda b, t: (0, 0)),
            pl.BlockSpec((C_OUT, 1), lambda b, t: (0, 0)),
        ],
        out_specs=[
            pl.BlockSpec((1, C_OUT, TN), lambda b, t: (b, 0, t)),
            pl.BlockSpec((1, 2, C_OUT), lambda b, t: (b * NT + t, 0, 0)),
        ],
        out_shape=[
            jax.ShapeDtypeStruct((Bn, C_OUT, N), jnp.float32),
            jax.ShapeDtypeStruct((Bn * NT, 2, C_OUT), jnp.float32),
        ],
    )(y1, a1, c1, W2, b2[:, None])

    a2, c2 = _scale_shift(st2, M, g2, beta2)
    out = pl.pallas_call(
        _bn_out_kernel,
        grid=grid,
        in_specs=[
            pl.BlockSpec((1, C_OUT, TN), lambda b, t: (b, 0, t)),
            pl.BlockSpec((C_OUT, 1), lambda b, t: (0, 0)),
            pl.BlockSpec((C_OUT, 1), lambda b, t: (0, 0)),
        ],
        out_specs=pl.BlockSpec((1, C_OUT, TN), lambda b, t: (b, 0, t)),
        out_shape=jax.ShapeDtypeStruct((Bn, C_OUT, N), jnp.float32),
    )(y2, a2, c2)
    return out


# speedup vs baseline: 32.2520x; 1.2540x over previous
import functools

import jax
import jax.numpy as jnp
from jax.experimental import pallas as pl


def _nn_interp_l1_kernel(xl_ref, xf_ref, ff_ref, fl_ref, w1a_ref, w1b_ref,
                         b1_ref, y1_ref, st_ref, *, S):
    xl = xl_ref[0]
    xf = xf_ref[0]
    nl = jnp.sum(xl * xl, axis=1, keepdims=True)
    nf = jnp.sum(xf * xf, axis=1)[None, :]
    d = -2.0 * jax.lax.dot_general(
        xl.astype(jnp.bfloat16), xf.astype(jnp.bfloat16),
        (((1,), (1,)), ((), ())),
        preferred_element_type=jnp.float32)
    d = d + nl
    d = d + nf
    dorig = d
    inf = jnp.float32(jnp.inf)
    slots = jnp.full((d.shape[0], 1), 3.0, jnp.float32)
    norm = jnp.zeros((d.shape[0], 1), jnp.float32)
    for _ in range(3):
        m = jnp.min(d, axis=1, keepdims=True)
        eq = d == m
        cnt = jnp.sum(jnp.where(eq, 1.0, 0.0), axis=1, keepdims=True)
        take = jnp.minimum(cnt, slots)
        w = 1.0 / (m + 1e-8)
        norm = norm + w * take
        d = jnp.where(eq & (slots > 0.0), inf, d)
        slots = slots - take
    wmat = jnp.where(d == inf, (1.0 / (dorig + 1e-8)) / norm, 0.0)
    interp = jnp.dot(wmat, ff_ref[0],
                     preferred_element_type=jnp.float32)
    y1 = jax.lax.dot_general(w1a_ref[...].astype(jnp.bfloat16),
                             fl_ref[0].astype(jnp.bfloat16),
                             (((1,), (1,)), ((), ())),
                             preferred_element_type=jnp.float32)
    y1 = y1 + jax.lax.dot_general(w1b_ref[...].astype(jnp.bfloat16),
                                  interp.astype(jnp.bfloat16),
                                  (((1,), (1,)), ((), ())),
                                  preferred_element_type=jnp.float32)
    y1 = y1 + b1_ref[...]
    y1_ref[0] = y1
    st_ref[0, 0, :] = jnp.sum(y1, axis=1)
    st_ref[0, 1, :] = jnp.sum(y1 * y1, axis=1)


def _bn_mm_kernel(y_ref, a_ref, c_ref, w2_ref, b2_ref, y2_ref, st_ref):
    h = jnp.maximum(y_ref[0] * a_ref[...] + c_ref[...], 0.0)
    y2 = jnp.dot(w2_ref[...].astype(jnp.bfloat16), h.astype(jnp.bfloat16),
                 preferred_element_type=jnp.float32) + b2_ref[...]
    y2_ref[0] = y2
    st_ref[0, 0, :] = jnp.sum(y2, axis=1)
    st_ref[0, 1, :] = jnp.sum(y2 * y2, axis=1)


def _bn_out_kernel(y_ref, a_ref, c_ref, o_ref):
    o_ref[0] = jnp.maximum(y_ref[0] * a_ref[...] + c_ref[...], 0.0)


def _scale_shift(st, m, gamma, beta):
    s = jnp.sum(st, axis=0)
    mean = s[0] / m
    var = s[1] / m - mean * mean
    inv = jax.lax.rsqrt(var + 1e-5)
    a = gamma * inv
    c = beta - mean * a
    return a[:, None], c[:, None]


def kernel(point_xyz_few, point_feature_few, point_xyz_large,
           point_feature_large, W1, b1, g1, beta1, W2, b2, g2, beta2):
    Bn, S, _ = point_xyz_few.shape
    N = point_xyz_large.shape[1]
    C_FEW = point_feature_few.shape[2]
    C_LARGE = point_feature_large.shape[2]
    C_MID = W1.shape[0]
    C_OUT = W2.shape[0]
    TN = 1024
    NT = N // TN
    M = Bn * N
    grid = (Bn, NT)

    y1, st1 = pl.pallas_call(
        functools.partial(_nn_interp_l1_kernel, S=S),
        grid=grid,
        in_specs=[
            pl.BlockSpec((1, TN, 3), lambda b, t: (b, t, 0)),
            pl.BlockSpec((1, S, 3), lambda b, t: (b, 0, 0)),
            pl.BlockSpec((1, S, C_FEW), lambda b, t: (b, 0, 0)),
            pl.BlockSpec((1, TN, C_LARGE), lambda b, t: (b, t, 0)),
            pl.BlockSpec((C_MID, C_LARGE), lambda b, t: (0, 0)),
            pl.BlockSpec((C_MID, C_FEW), lambda b, t: (0, 0)),
            pl.BlockSpec((C_MID, 1), lambda b, t: (0, 0)),
        ],
        out_specs=[
            pl.BlockSpec((1, C_MID, TN), lambda b, t: (b, 0, t)),
            pl.BlockSpec((1, 2, C_MID), lambda b, t: (b * NT + t, 0, 0)),
        ],
        out_shape=[
            jax.ShapeDtypeStruct((Bn, C_MID, N), jnp.float32),
            jax.ShapeDtypeStruct((Bn * NT, 2, C_MID), jnp.float32),
        ],
    )(point_xyz_large, point_xyz_few, point_feature_few, point_feature_large,
      W1[:, :C_LARGE], W1[:, C_LARGE:], b1[:, None])

    a1, c1 = _scale_shift(st1, M, g1, beta1)
    y2, st2 = pl.pallas_call(
        _bn_mm_kernel,
        grid=grid,
        in_specs=[
            pl.BlockSpec((1, C_MID, TN), lambda b, t: (b, 0, t)),
            pl.BlockSpec((C_MID, 1), lambda b, t: (0, 0)),
            pl.BlockSpec((C_MID, 1), lambda b, t: (0, 0)),
            pl.BlockSpec((C_OUT, C_MID), lambda b, t: (0, 0)),
            pl.BlockSpec((C_OUT, 1), lambda b, t: (0, 0)),
        ],
        out_specs=[
            pl.BlockSpec((1, C_OUT, TN), lambda b, t: (b, 0, t)),
            pl.BlockSpec((1, 2, C_OUT), lambda b, t: (b * NT + t, 0, 0)),
        ],
        out_shape=[
            jax.ShapeDtypeStruct((Bn, C_OUT, N), jnp.float32),
            jax.ShapeDtypeStruct((Bn * NT, 2, C_OUT), jnp.float32),
        ],
    )(y1, a1, c1, W2, b2[:, None])

    a2, c2 = _scale_shift(st2, M, g2, beta2)
    out = pl.pallas_call(
        _bn_out_kernel,
        grid=grid,
        in_specs=[
            pl.BlockSpec((1, C_OUT, TN), lambda b, t: (b, 0, t)),
            pl.BlockSpec((C_OUT, 1), lambda b, t: (0, 0)),
            pl.BlockSpec((C_OUT, 1), lambda b, t: (0, 0)),
        ],
        out_specs=pl.BlockSpec((1, C_OUT, TN), lambda b, t: (b, 0, t)),
        out_shape=jax.ShapeDtypeStruct((Bn, C_OUT, N), jnp.float32),
    )(y2, a2, c2)
    return out


# speedup vs baseline: 37.2190x; 1.1540x over previous
import functools

import jax
import jax.numpy as jnp
from jax.experimental import pallas as pl


def _nn_interp_l1_kernel(xl_ref, xf_ref, ff_ref, fl_ref, w1a_ref, w1b_ref,
                         b1_ref, y1_ref, st_ref, *, S):
    xl = xl_ref[0]
    xf = xf_ref[0]
    nl = jnp.sum(xl * xl, axis=1, keepdims=True)
    nf = jnp.sum(xf * xf, axis=1)[None, :]
    d = -2.0 * jax.lax.dot_general(
        xl.astype(jnp.bfloat16), xf.astype(jnp.bfloat16),
        (((1,), (1,)), ((), ())),
        preferred_element_type=jnp.float32)
    d = d + nl
    d = d + nf
    dorig = d
    inf = jnp.float32(jnp.inf)
    slots = jnp.full((d.shape[0], 1), 3.0, jnp.float32)
    norm = jnp.zeros((d.shape[0], 1), jnp.float32)
    for _ in range(3):
        m = jnp.min(d, axis=1, keepdims=True)
        eq = d == m
        cnt = jnp.sum(jnp.where(eq, 1.0, 0.0), axis=1, keepdims=True)
        take = jnp.minimum(cnt, slots)
        w = 1.0 / (m + 1e-8)
        norm = norm + w * take
        d = jnp.where(eq & (slots > 0.0), inf, d)
        slots = slots - take
    wmat = jnp.where(d == inf, (1.0 / (dorig + 1e-8)) / norm, 0.0)
    interp = jnp.dot(wmat, ff_ref[0],
                     preferred_element_type=jnp.float32)
    y1 = jax.lax.dot_general(w1a_ref[...].astype(jnp.bfloat16),
                             fl_ref[0].astype(jnp.bfloat16),
                             (((1,), (1,)), ((), ())),
                             preferred_element_type=jnp.float32)
    y1 = y1 + jax.lax.dot_general(w1b_ref[...].astype(jnp.bfloat16),
                                  interp.astype(jnp.bfloat16),
                                  (((1,), (1,)), ((), ())),
                                  preferred_element_type=jnp.float32)
    y1 = y1 + b1_ref[...]
    y1_ref[0] = y1
    st_ref[0, 0, :] = jnp.sum(y1, axis=1)
    st_ref[0, 1, :] = jnp.sum(y1 * y1, axis=1)


def _bn_mm_kernel(y_ref, a_ref, c_ref, w2_ref, b2_ref, y2_ref, st_ref):
    h = jnp.maximum(y_ref[0] * a_ref[...] + c_ref[...], 0.0)
    y2 = jnp.dot(w2_ref[...].astype(jnp.bfloat16), h.astype(jnp.bfloat16),
                 preferred_element_type=jnp.float32) + b2_ref[...]
    y2_ref[0] = y2
    st_ref[0, 0, :] = jnp.sum(y2, axis=1)
    st_ref[0, 1, :] = jnp.sum(y2 * y2, axis=1)


def _bn_out_kernel(y_ref, a_ref, c_ref, o_ref):
    o_ref[0] = jnp.maximum(y_ref[0] * a_ref[...] + c_ref[...], 0.0)


def _scale_shift(st, m, gamma, beta):
    s = jnp.sum(st, axis=0)
    mean = s[0] / m
    var = s[1] / m - mean * mean
    inv = jax.lax.rsqrt(var + 1e-5)
    a = gamma * inv
    c = beta - mean * a
    return a[:, None], c[:, None]


def kernel(point_xyz_few, point_feature_few, point_xyz_large,
           point_feature_large, W1, b1, g1, beta1, W2, b2, g2, beta2):
    Bn, S, _ = point_xyz_few.shape
    N = point_xyz_large.shape[1]
    C_FEW = point_feature_few.shape[2]
    C_LARGE = point_feature_large.shape[2]
    C_MID = W1.shape[0]
    C_OUT = W2.shape[0]
    TN = 2048
    NT = N // TN
    M = Bn * N
    grid = (Bn, NT)

    y1, st1 = pl.pallas_call(
        functools.partial(_nn_interp_l1_kernel, S=S),
        grid=grid,
        in_specs=[
            pl.BlockSpec((1, TN, 3), lambda b, t: (b, t, 0)),
            pl.BlockSpec((1, S, 3), lambda b, t: (b, 0, 0)),
            pl.BlockSpec((1, S, C_FEW), lambda b, t: (b, 0, 0)),
            pl.BlockSpec((1, TN, C_LARGE), lambda b, t: (b, t, 0)),
            pl.BlockSpec((C_MID, C_LARGE), lambda b, t: (0, 0)),
            pl.BlockSpec((C_MID, C_FEW), lambda b, t: (0, 0)),
            pl.BlockSpec((C_MID, 1), lambda b, t: (0, 0)),
        ],
        out_specs=[
            pl.BlockSpec((1, C_MID, TN), lambda b, t: (b, 0, t)),
            pl.BlockSpec((1, 2, C_MID), lambda b, t: (b * NT + t, 0, 0)),
        ],
        out_shape=[
            jax.ShapeDtypeStruct((Bn, C_MID, N), jnp.float32),
            jax.ShapeDtypeStruct((Bn * NT, 2, C_MID), jnp.float32),
        ],
    )(point_xyz_large, point_xyz_few, point_feature_few, point_feature_large,
      W1[:, :C_LARGE], W1[:, C_LARGE:], b1[:, None])

    a1, c1 = _scale_shift(st1, M, g1, beta1)
    y2, st2 = pl.pallas_call(
        _bn_mm_kernel,
        grid=grid,
        in_specs=[
            pl.BlockSpec((1, C_MID, TN), lambda b, t: (b, 0, t)),
            pl.BlockSpec((C_MID, 1), lambda b, t: (0, 0)),
            pl.BlockSpec((C_MID, 1), lambda b, t: (0, 0)),
            pl.BlockSpec((C_OUT, C_MID), lambda b, t: (0, 0)),
            pl.BlockSpec((C_OUT, 1), lambda b, t: (0, 0)),
        ],
        out_specs=[
            pl.BlockSpec((1, C_OUT, TN), lambda b, t: (b, 0, t)),
            pl.BlockSpec((1, 2, C_OUT), lambda b, t: (b * NT + t, 0, 0)),
        ],
        out_shape=[
            jax.ShapeDtypeStruct((Bn, C_OUT, N), jnp.float32),
            jax.ShapeDtypeStruct((Bn * NT, 2, C_OUT), jnp.float32),
        ],
    )(y1, a1, c1, W2, b2[:, None])

    a2, c2 = _scale_shift(st2, M, g2, beta2)
    out = pl.pallas_call(
        _bn_out_kernel,
        grid=grid,
        in_specs=[
            pl.BlockSpec((1, C_OUT, TN), lambda b, t: (b, 0, t)),
            pl.BlockSpec((C_OUT, 1), lambda b, t: (0, 0)),
            pl.BlockSpec((C_OUT, 1), lambda b, t: (0, 0)),
        ],
        out_specs=pl.BlockSpec((1, C_OUT, TN), lambda b, t: (b, 0, t)),
        out_shape=jax.ShapeDtypeStruct((Bn, C_OUT, N), jnp.float32),
    )(y2, a2, c2)
    return out


# speedup vs baseline: 39.4975x; 1.0612x over previous
import functools

import jax
import jax.numpy as jnp
from jax.experimental import pallas as pl


def _nn_interp_l1_kernel(xl_ref, xf_ref, ff_ref, fl_ref, w1a_ref, w1b_ref,
                         b1_ref, y1_ref, st_ref, *, S):
    xl = xl_ref[0]
    xf = xf_ref[0]
    nl = jnp.sum(xl * xl, axis=1, keepdims=True)
    nf = jnp.sum(xf * xf, axis=1)[None, :]
    d = -2.0 * jax.lax.dot_general(
        xl.astype(jnp.bfloat16), xf.astype(jnp.bfloat16),
        (((1,), (1,)), ((), ())),
        preferred_element_type=jnp.float32)
    d = d + nl
    d = d + nf
    dorig = d
    inf = jnp.float32(jnp.inf)
    slots = jnp.full((d.shape[0], 1), 3.0, jnp.float32)
    norm = jnp.zeros((d.shape[0], 1), jnp.float32)
    for _ in range(3):
        m = jnp.min(d, axis=1, keepdims=True)
        eq = d == m
        cnt = jnp.sum(jnp.where(eq, 1.0, 0.0), axis=1, keepdims=True)
        take = jnp.minimum(cnt, slots)
        w = 1.0 / (m + 1e-8)
        norm = norm + w * take
        d = jnp.where(eq & (slots > 0.0), inf, d)
        slots = slots - take
    wmat = jnp.where(d == inf, (1.0 / (dorig + 1e-8)) / norm, 0.0)
    interp = jnp.dot(wmat, ff_ref[0],
                     preferred_element_type=jnp.float32)
    y1 = jax.lax.dot_general(w1a_ref[...].astype(jnp.bfloat16),
                             fl_ref[0].astype(jnp.bfloat16),
                             (((1,), (1,)), ((), ())),
                             preferred_element_type=jnp.float32)
    y1 = y1 + jax.lax.dot_general(w1b_ref[...].astype(jnp.bfloat16),
                                  interp.astype(jnp.bfloat16),
                                  (((1,), (1,)), ((), ())),
                                  preferred_element_type=jnp.float32)
    y1 = y1 + b1_ref[...]
    y1_ref[0] = y1
    st_ref[0, 0, :] = jnp.sum(y1, axis=1)
    st_ref[0, 1, :] = jnp.sum(y1 * y1, axis=1)


def _bn_mm_kernel(y_ref, a_ref, c_ref, w2_ref, b2_ref, y2_ref, st_ref):
    h = jnp.maximum(y_ref[0] * a_ref[...] + c_ref[...], 0.0)
    y2 = jnp.dot(w2_ref[...].astype(jnp.bfloat16), h.astype(jnp.bfloat16),
                 preferred_element_type=jnp.float32) + b2_ref[...]
    y2_ref[0] = y2
    st_ref[0, 0, :] = jnp.sum(y2, axis=1)
    st_ref[0, 1, :] = jnp.sum(y2 * y2, axis=1)


def _bn_out_kernel(y_ref, a_ref, c_ref, o_ref):
    o_ref[0] = jnp.maximum(y_ref[0] * a_ref[...] + c_ref[...], 0.0)


def _scale_shift(st, m, gamma, beta):
    s = jnp.sum(st, axis=0)
    mean = s[0] / m
    var = s[1] / m - mean * mean
    inv = jax.lax.rsqrt(var + 1e-5)
    a = gamma * inv
    c = beta - mean * a
    return a[:, None], c[:, None]


def kernel(point_xyz_few, point_feature_few, point_xyz_large,
           point_feature_large, W1, b1, g1, beta1, W2, b2, g2, beta2):
    Bn, S, _ = point_xyz_few.shape
    N = point_xyz_large.shape[1]
    C_FEW = point_feature_few.shape[2]
    C_LARGE = point_feature_large.shape[2]
    C_MID = W1.shape[0]
    C_OUT = W2.shape[0]
    TN = 4096
    NT = N // TN
    M = Bn * N
    grid = (Bn, NT)

    y1, st1 = pl.pallas_call(
        functools.partial(_nn_interp_l1_kernel, S=S),
        grid=grid,
        in_specs=[
            pl.BlockSpec((1, TN, 3), lambda b, t: (b, t, 0)),
            pl.BlockSpec((1, S, 3), lambda b, t: (b, 0, 0)),
            pl.BlockSpec((1, S, C_FEW), lambda b, t: (b, 0, 0)),
            pl.BlockSpec((1, TN, C_LARGE), lambda b, t: (b, t, 0)),
            pl.BlockSpec((C_MID, C_LARGE), lambda b, t: (0, 0)),
            pl.BlockSpec((C_MID, C_FEW), lambda b, t: (0, 0)),
            pl.BlockSpec((C_MID, 1), lambda b, t: (0, 0)),
        ],
        out_specs=[
            pl.BlockSpec((1, C_MID, TN), lambda b, t: (b, 0, t)),
            pl.BlockSpec((1, 2, C_MID), lambda b, t: (b * NT + t, 0, 0)),
        ],
        out_shape=[
            jax.ShapeDtypeStruct((Bn, C_MID, N), jnp.float32),
            jax.ShapeDtypeStruct((Bn * NT, 2, C_MID), jnp.float32),
        ],
    )(point_xyz_large, point_xyz_few, point_feature_few, point_feature_large,
      W1[:, :C_LARGE], W1[:, C_LARGE:], b1[:, None])

    a1, c1 = _scale_shift(st1, M, g1, beta1)
    y2, st2 = pl.pallas_call(
        _bn_mm_kernel,
        grid=grid,
        in_specs=[
            pl.BlockSpec((1, C_MID, TN), lambda b, t: (b, 0, t)),
            pl.BlockSpec((C_MID, 1), lambda b, t: (0, 0)),
            pl.BlockSpec((C_MID, 1), lambda b, t: (0, 0)),
            pl.BlockSpec((C_OUT, C_MID), lambda b, t: (0, 0)),
            pl.BlockSpec((C_OUT, 1), lambda b, t: (0, 0)),
        ],
        out_specs=[
            pl.BlockSpec((1, C_OUT, TN), lambda b, t: (b, 0, t)),
            pl.BlockSpec((1, 2, C_OUT), lambda b, t: (b * NT + t, 0, 0)),
        ],
        out_shape=[
            jax.ShapeDtypeStruct((Bn, C_OUT, N), jnp.float32),
            jax.ShapeDtypeStruct((Bn * NT, 2, C_OUT), jnp.float32),
        ],
    )(y1, a1, c1, W2, b2[:, None])

    a2, c2 = _scale_shift(st2, M, g2, beta2)
    out = pl.pallas_call(
        _bn_out_kernel,
        grid=grid,
        in_specs=[
            pl.BlockSpec((1, C_OUT, TN), lambda b, t: (b, 0, t)),
            pl.BlockSpec((C_OUT, 1), lambda b, t: (0, 0)),
            pl.BlockSpec((C_OUT, 1), lambda b, t: (0, 0)),
        ],
        out_specs=pl.BlockSpec((1, C_OUT, TN), lambda b, t: (b, 0, t)),
        out_shape=jax.ShapeDtypeStruct((Bn, C_OUT, N), jnp.float32),
    )(y2, a2, c2)
    return out


# speedup vs baseline: 46.3486x; 1.1735x over previous
import functools

import jax
import jax.numpy as jnp
from jax.experimental import pallas as pl


def _nn_interp_l1_kernel(xl_ref, xf_ref, ff_ref, fl_ref, w1a_ref, w1b_ref,
                         b1_ref, y1_ref, st_ref, *, S):
    xl = xl_ref[0]
    xf = xf_ref[0]
    nl = jnp.sum(xl * xl, axis=1, keepdims=True)
    nf = jnp.sum(xf * xf, axis=1)[None, :]
    d = -2.0 * jax.lax.dot_general(
        xl.astype(jnp.bfloat16), xf.astype(jnp.bfloat16),
        (((1,), (1,)), ((), ())),
        preferred_element_type=jnp.float32)
    d = d + nl
    d = d + nf
    inf = jnp.float32(jnp.inf)
    h = d.shape[1] // 2
    a, b = jnp.minimum(d[:, :h], d[:, h:]), jnp.maximum(d[:, :h], d[:, h:])
    c = None
    while h > 128:
        h //= 2
        a1, a2 = a[:, :h], a[:, h:]
        b1, b2 = b[:, :h], b[:, h:]
        h1 = jnp.maximum(a1, a2)
        l2 = jnp.minimum(b1, b2)
        a = jnp.minimum(a1, a2)
        if c is None:
            b, c = jnp.minimum(h1, l2), jnp.maximum(h1, l2)
        else:
            l3 = jnp.minimum(c[:, :h], c[:, h:])
            b = jnp.minimum(h1, l2)
            c = jnp.minimum(jnp.maximum(h1, l2), l3)
    u = jnp.concatenate([a, b] + ([c] if c is not None else []), axis=1)
    slots = jnp.full((u.shape[0], 1), 3.0, jnp.float32)
    norm = jnp.zeros((u.shape[0], 1), jnp.float32)
    thr = jnp.full((u.shape[0], 1), inf, jnp.float32)
    for _ in range(3):
        m = jnp.min(u, axis=1, keepdims=True)
        eq = u == m
        cnt = jnp.sum(jnp.where(eq, 1.0, 0.0), axis=1, keepdims=True)
        take = jnp.minimum(cnt, slots)
        w = 1.0 / (m + 1e-8)
        norm = norm + w * take
        live = slots > 0.0
        thr = jnp.where(live, m, thr)
        u = jnp.where(eq & live, inf, u)
        slots = slots - take
    wmat = jnp.where(d <= thr, 1.0 / (d + 1e-8), 0.0)
    interp = jnp.dot(wmat, ff_ref[0],
                     preferred_element_type=jnp.float32) / norm
    y1 = jax.lax.dot_general(w1a_ref[...].astype(jnp.bfloat16),
                             fl_ref[0].astype(jnp.bfloat16),
                             (((1,), (1,)), ((), ())),
                             preferred_element_type=jnp.float32)
    y1 = y1 + jax.lax.dot_general(w1b_ref[...].astype(jnp.bfloat16),
                                  interp.astype(jnp.bfloat16),
                                  (((1,), (1,)), ((), ())),
                                  preferred_element_type=jnp.float32)
    y1 = y1 + b1_ref[...]
    y1_ref[0] = y1
    st_ref[0, 0, :] = jnp.sum(y1, axis=1)
    st_ref[0, 1, :] = jnp.sum(y1 * y1, axis=1)


def _bn_mm_kernel(y_ref, a_ref, c_ref, w2_ref, b2_ref, y2_ref, st_ref):
    h = jnp.maximum(y_ref[0] * a_ref[...] + c_ref[...], 0.0)
    y2 = jnp.dot(w2_ref[...].astype(jnp.bfloat16), h.astype(jnp.bfloat16),
                 preferred_element_type=jnp.float32) + b2_ref[...]
    y2_ref[0] = y2
    st_ref[0, 0, :] = jnp.sum(y2, axis=1)
    st_ref[0, 1, :] = jnp.sum(y2 * y2, axis=1)


def _bn_out_kernel(y_ref, a_ref, c_ref, o_ref):
    o_ref[0] = jnp.maximum(y_ref[0] * a_ref[...] + c_ref[...], 0.0)


def _scale_shift(st, m, gamma, beta):
    s = jnp.sum(st, axis=0)
    mean = s[0] / m
    var = s[1] / m - mean * mean
    inv = jax.lax.rsqrt(var + 1e-5)
    a = gamma * inv
    c = beta - mean * a
    return a[:, None], c[:, None]


def kernel(point_xyz_few, point_feature_few, point_xyz_large,
           point_feature_large, W1, b1, g1, beta1, W2, b2, g2, beta2):
    Bn, S, _ = point_xyz_few.shape
    N = point_xyz_large.shape[1]
    C_FEW = point_feature_few.shape[2]
    C_LARGE = point_feature_large.shape[2]
    C_MID = W1.shape[0]
    C_OUT = W2.shape[0]
    TN = min(4096, N)
    NT = N // TN
    M = Bn * N
    grid = (Bn, NT)

    y1, st1 = pl.pallas_call(
        functools.partial(_nn_interp_l1_kernel, S=S),
        grid=grid,
        in_specs=[
            pl.BlockSpec((1, TN, 3), lambda b, t: (b, t, 0)),
            pl.BlockSpec((1, S, 3), lambda b, t: (b, 0, 0)),
            pl.BlockSpec((1, S, C_FEW), lambda b, t: (b, 0, 0)),
            pl.BlockSpec((1, TN, C_LARGE), lambda b, t: (b, t, 0)),
            pl.BlockSpec((C_MID, C_LARGE), lambda b, t: (0, 0)),
            pl.BlockSpec((C_MID, C_FEW), lambda b, t: (0, 0)),
            pl.BlockSpec((C_MID, 1), lambda b, t: (0, 0)),
        ],
        out_specs=[
            pl.BlockSpec((1, C_MID, TN), lambda b, t: (b, 0, t)),
            pl.BlockSpec((1, 2, C_MID), lambda b, t: (b * NT + t, 0, 0)),
        ],
        out_shape=[
            jax.ShapeDtypeStruct((Bn, C_MID, N), jnp.float32),
            jax.ShapeDtypeStruct((Bn * NT, 2, C_MID), jnp.float32),
        ],
    )(point_xyz_large, point_xyz_few, point_feature_few, point_feature_large,
      W1[:, :C_LARGE], W1[:, C_LARGE:], b1[:, None])

    a1, c1 = _scale_shift(st1, M, g1, beta1)
    y2, st2 = pl.pallas_call(
        _bn_mm_kernel,
        grid=grid,
        in_specs=[
            pl.BlockSpec((1, C_MID, TN), lambda b, t: (b, 0, t)),
            pl.BlockSpec((C_MID, 1), lambda b, t: (0, 0)),
            pl.BlockSpec((C_MID, 1), lambda b, t: (0, 0)),
            pl.BlockSpec((C_OUT, C_MID), lambda b, t: (0, 0)),
            pl.BlockSpec((C_OUT, 1), lambda b, t: (0, 0)),
        ],
        out_specs=[
            pl.BlockSpec((1, C_OUT, TN), lambda b, t: (b, 0, t)),
            pl.BlockSpec((1, 2, C_OUT), lambda b, t: (b * NT + t, 0, 0)),
        ],
        out_shape=[
            jax.ShapeDtypeStruct((Bn, C_OUT, N), jnp.float32),
            jax.ShapeDtypeStruct((Bn * NT, 2, C_OUT), jnp.float32),
        ],
    )(y1, a1, c1, W2, b2[:, None])

    a2, c2 = _scale_shift(st2, M, g2, beta2)
    out = pl.pallas_call(
        _bn_out_kernel,
        grid=grid,
        in_specs=[
            pl.BlockSpec((1, C_OUT, TN), lambda b, t: (b, 0, t)),
            pl.BlockSpec((C_OUT, 1), lambda b, t: (0, 0)),
            pl.BlockSpec((C_OUT, 1), lambda b, t: (0, 0)),
        ],
        out_specs=pl.BlockSpec((1, C_OUT, TN), lambda b, t: (b, 0, t)),
        out_shape=jax.ShapeDtypeStruct((Bn, C_OUT, N), jnp.float32),
    )(y2, a2, c2)
    return out
